# Initial kernel scaffold; baseline (speedup 1.0000x reference)
import jax
import jax.numpy as jnp
from jax.experimental import pallas as pl


def kernel(x, l0_W1, l0_b1, l0_gamma, l0_beta, l0_W2, l0_b2, l1_W1, l1_b1, l1_gamma, l1_beta, l1_W2, l1_b2, l2_W1, l2_b1, l2_gamma, l2_beta, l2_W2, l2_b2, jk_W, jk_b, c1_W, c1_b, c_gamma, c_beta, c2_W, c2_b, edge_index, batch):
    raise NotImplementedError("write your pallas kernel here")


# speedup vs baseline: 4.1360x; 4.1360x over previous
import functools

import jax
import jax.numpy as jnp
from jax import lax
from jax.experimental import pallas as pl
from jax.experimental.pallas import tpu as pltpu
from jax.experimental.pallas import tpu_sc as plsc

N = 10000
E = 320000
D = 128
H = 64
G = 128
C = 40

NC = 2
NS = 16
NW = NC * NS

LANE = 128
EPW = E // NW
SPW = 82
CAP = SPW * LANE
N_PAD = 10112


def _make_seg_sum(F):
    rps = N_PAD // NS

    mesh = plsc.VectorSubcoreMesh(core_axis_name="c", subcore_axis_name="s")

    @functools.partial(
        pl.kernel,
        out_type=jax.ShapeDtypeStruct((NC * N_PAD, F), jnp.float32),
        mesh=mesh,
        compiler_params=pltpu.CompilerParams(use_tc_tiling_on_sc=False),
        scratch_types=[
            pltpu.VMEM((SPW, LANE), jnp.int32),
            pltpu.VMEM((SPW, LANE), jnp.int32),
            pltpu.VMEM((LANE, F), jnp.float32),
            pltpu.VMEM_SHARED((N_PAD, F), jnp.float32),
            pltpu.SemaphoreType.DMA,
        ],
    )
    def seg(x_hbm, src_hbm, dst_hbm, zeros_hbm, out_hbm, src_v, dst_v, rows_v,
            acc, sem):
        c = lax.axis_index("c")
        s = lax.axis_index("s")
        wid = s * NC + c
        pltpu.sync_copy(zeros_hbm.at[pl.ds(s * rps, rps)],
                        acc.at[pl.ds(s * rps, rps)])
        plsc.subcore_barrier()
        pltpu.sync_copy(src_hbm.at[wid], src_v)
        pltpu.sync_copy(dst_hbm.at[wid], dst_v)

        def body(j, carry):
            pltpu.async_copy(x_hbm.at[src_v.at[j]], rows_v, sem).wait()
            pltpu.sync_copy(rows_v, acc.at[dst_v.at[j]], add=True)
            return carry

        lax.fori_loop(0, SPW, body, 0)
        plsc.subcore_barrier()
        pltpu.sync_copy(acc.at[pl.ds(s * rps, rps)],
                        out_hbm.at[pl.ds(c * N_PAD + s * rps, rps)])

    return seg


_seg_sum_128 = _make_seg_sum(D)
_seg_sum_64 = _make_seg_sum(H)


def _gin_mlp(x, parts, W1, b1, g, bt, W2, b2):
    Dout = W2.shape[1]

    def body(x_ref, p_ref, W1_ref, b1_ref, g_ref, bt_ref, W2_ref, b2_ref, o_ref):
        h = x_ref[...] + p_ref[0:N, :] + p_ref[N_PAD:N_PAD + N, :]
        h = jnp.dot(h, W1_ref[...], preferred_element_type=jnp.float32)
        h = h + b1_ref[0, :]
        m = jnp.sum(h, axis=0, keepdims=True) * jnp.float32(1e-4)
        d = h - m
        v = jnp.sum(d * d, axis=0, keepdims=True) * jnp.float32(1e-4)
        h = d / jnp.sqrt(v + 1e-5) * g_ref[0, :] + bt_ref[0, :]
        h = jnp.maximum(h, 0.0)
        h = jnp.dot(h, W2_ref[...], preferred_element_type=jnp.float32)
        h = h + b2_ref[0, :]
        o_ref[...] = jnp.maximum(h, 0.0)

    return pl.pallas_call(
        body,
        out_shape=jax.ShapeDtypeStruct((N, Dout), jnp.float32),
    )(x, parts, W1, b1.reshape(1, -1), g.reshape(1, -1), bt.reshape(1, -1),
      W2, b2.reshape(1, -1))


def _head(x1, x2, x3, jk_W, jk_b, batch2d, c1_W, c1_b, cg, cb, c2_W, c2_b):
    def body(x1_ref, x2_ref, x3_ref, jkW_ref, jkb_ref, b_ref, c1W_ref,
             c1b_ref, cg_ref, cb_ref, c2W_ref, c2b_ref, o_ref):
        hcat = jnp.concatenate([x1_ref[...], x2_ref[...], x3_ref[...]], axis=1)
        h = jnp.dot(hcat, jkW_ref[...], preferred_element_type=jnp.float32)
        h = h + jkb_ref[0, :]
        ids = b_ref[0, :]
        onehot = (lax.broadcasted_iota(jnp.int32, (G, N), 0) == ids[None, :])
        pooled = lax.dot(onehot.astype(jnp.float32), h,
                         precision=lax.Precision.HIGHEST,
                         preferred_element_type=jnp.float32)
        z = jnp.dot(pooled, c1W_ref[...], preferred_element_type=jnp.float32)
        z = z + c1b_ref[0, :]
        m = jnp.sum(z, axis=0, keepdims=True) * jnp.float32(1.0 / G)
        d = z - m
        v = jnp.sum(d * d, axis=0, keepdims=True) * jnp.float32(1.0 / G)
        z = d / jnp.sqrt(v + 1e-5) * cg_ref[0, :] + cb_ref[0, :]
        z = jnp.maximum(z, 0.0)
        z = jnp.dot(z, c2W_ref[...], preferred_element_type=jnp.float32)
        o_ref[...] = z + c2b_ref[0, :]

    return pl.pallas_call(
        body,
        out_shape=jax.ShapeDtypeStruct((G, C), jnp.float32),
    )(x1, x2, x3, jk_W, jk_b.reshape(1, -1), batch2d, c1_W,
      c1_b.reshape(1, -1), cg.reshape(1, -1), cb.reshape(1, -1), c2_W,
      c2_b.reshape(1, -1))


def kernel(x, l0_W1, l0_b1, l0_gamma, l0_beta, l0_W2, l0_b2,
           l1_W1, l1_b1, l1_gamma, l1_beta, l1_W2, l1_b2,
           l2_W1, l2_b1, l2_gamma, l2_beta, l2_W2, l2_b2,
           jk_W, jk_b, c1_W, c1_b, c_gamma, c_beta, c2_W, c2_b,
           edge_index, batch):
    src = edge_index[0].astype(jnp.int32)
    dst = edge_index[1].astype(jnp.int32)
    order = jnp.argsort(dst, stable=True).astype(jnp.int32)
    dst_s = dst[order]
    src_s = src[order]
    bnd = jnp.arange(1, NW, dtype=jnp.int32) * EPW
    vals = dst_s[bnd]
    left = jnp.searchsorted(dst_s, vals, side='left').astype(jnp.int32)
    right = jnp.searchsorted(dst_s, vals, side='right').astype(jnp.int32)
    mid = jnp.where(left < bnd, right, bnd)
    starts = jnp.concatenate([jnp.zeros((1,), jnp.int32), mid,
                              jnp.full((1,), E, jnp.int32)])
    lanes = jnp.arange(CAP, dtype=jnp.int32)
    pos = starts[:NW, None] + lanes[None, :]
    valid = pos < starts[1:, None]
    posc = jnp.minimum(pos, E - 1)
    src_r = jnp.where(valid, src_s[posc], lanes[None, :] % N)
    dst_r = jnp.where(valid, dst_s[posc], N + (lanes[None, :] % (N_PAD - N)))
    src_r = src_r.reshape(NW, SPW, LANE)
    dst_r = dst_r.reshape(NW, SPW, LANE)
    zeros128 = jnp.zeros((N_PAD, D), jnp.float32)
    zeros64 = jnp.zeros((N_PAD, H), jnp.float32)
    batch2d = batch.astype(jnp.int32).reshape(1, N)

    parts0 = _seg_sum_128(x, src_r, dst_r, zeros128)
    x1 = _gin_mlp(x, parts0, l0_W1, l0_b1, l0_gamma, l0_beta, l0_W2, l0_b2)
    parts1 = _seg_sum_64(x1, src_r, dst_r, zeros64)
    x2 = _gin_mlp(x1, parts1, l1_W1, l1_b1, l1_gamma, l1_beta, l1_W2, l1_b2)
    parts2 = _seg_sum_64(x2, src_r, dst_r, zeros64)
    x3 = _gin_mlp(x2, parts2, l2_W1, l2_b1, l2_gamma, l2_beta, l2_W2, l2_b2)
    return _head(x1, x2, x3, jk_W, jk_b, batch2d, c1_W, c1_b, c_gamma,
                 c_beta, c2_W, c2_b)


# speedup vs baseline: 4.7696x; 1.1532x over previous
import functools

import jax
import jax.numpy as jnp
from jax import lax
from jax.experimental import pallas as pl
from jax.experimental.pallas import tpu as pltpu
from jax.experimental.pallas import tpu_sc as plsc

N = 10000
E = 320000
D = 128
H = 64
G = 128
C = 40

NC = 2
NS = 16
NW = NC * NS

LANE = 128
EPW = E // NW
SPW = 82
CAP = SPW * LANE
N_PAD = 10112


def _make_seg_sum(F):
    rps = N_PAD // NS
    LANE_F = 64 if F == 128 else 128
    SPW_F = CAP // LANE_F

    mesh = plsc.VectorSubcoreMesh(core_axis_name="c", subcore_axis_name="s")

    @functools.partial(
        pl.kernel,
        out_type=jax.ShapeDtypeStruct((NC * N_PAD, F), jnp.float32),
        mesh=mesh,
        compiler_params=pltpu.CompilerParams(use_tc_tiling_on_sc=False),
        scratch_types=[
            pltpu.VMEM((SPW_F, LANE_F), jnp.int32),
            pltpu.VMEM((SPW_F, LANE_F), jnp.int32),
            pltpu.VMEM((LANE_F, F), jnp.float32),
            pltpu.VMEM((LANE_F, F), jnp.float32),
            pltpu.VMEM_SHARED((N_PAD, F), jnp.float32),
            pltpu.SemaphoreType.DMA,
            pltpu.SemaphoreType.DMA,
        ],
    )
    def seg(x_hbm, src_hbm, dst_hbm, zeros_hbm, out_hbm, src_v, dst_v, rows0,
            rows1, acc, sem0, sem1):
        c = lax.axis_index("c")
        s = lax.axis_index("s")
        wid = s * NC + c
        pltpu.sync_copy(zeros_hbm.at[pl.ds(s * rps, rps)],
                        acc.at[pl.ds(s * rps, rps)])
        plsc.subcore_barrier()
        pltpu.sync_copy(src_hbm.at[wid], src_v)
        pltpu.sync_copy(dst_hbm.at[wid], dst_v)

        pltpu.async_copy(x_hbm.at[src_v.at[0]], rows0, sem0)
        pltpu.async_copy(x_hbm.at[src_v.at[1]], rows1, sem1)

        def body(i, carry):
            j0 = 2 * i
            j1 = j0 + 1
            pltpu.make_async_copy(x_hbm.at[src_v.at[j0]], rows0, sem0).wait()
            pltpu.sync_copy(rows0, acc.at[dst_v.at[j0]], add=True)

            @pl.when(i < SPW_F // 2 - 1)
            def _():
                pltpu.async_copy(x_hbm.at[src_v.at[j0 + 2]], rows0, sem0)

            pltpu.make_async_copy(x_hbm.at[src_v.at[j1]], rows1, sem1).wait()
            pltpu.sync_copy(rows1, acc.at[dst_v.at[j1]], add=True)

            @pl.when(i < SPW_F // 2 - 1)
            def _():
                pltpu.async_copy(x_hbm.at[src_v.at[j1 + 2]], rows1, sem1)

            return carry

        lax.fori_loop(0, SPW_F // 2, body, 0)
        plsc.subcore_barrier()
        pltpu.sync_copy(acc.at[pl.ds(s * rps, rps)],
                        out_hbm.at[pl.ds(c * N_PAD + s * rps, rps)])

    return seg


_seg_sum_128 = _make_seg_sum(D)
_seg_sum_64 = _make_seg_sum(H)


def _gin_mlp(x, parts, W1, b1, g, bt, W2, b2):
    Dout = W2.shape[1]

    def body(x_ref, p_ref, W1_ref, b1_ref, g_ref, bt_ref, W2_ref, b2_ref, o_ref):
        h = x_ref[...] + p_ref[0:N, :] + p_ref[N_PAD:N_PAD + N, :]
        h = jnp.dot(h, W1_ref[...], preferred_element_type=jnp.float32)
        h = h + b1_ref[0, :]
        m = jnp.sum(h, axis=0, keepdims=True) * jnp.float32(1e-4)
        d = h - m
        v = jnp.sum(d * d, axis=0, keepdims=True) * jnp.float32(1e-4)
        h = d / jnp.sqrt(v + 1e-5) * g_ref[0, :] + bt_ref[0, :]
        h = jnp.maximum(h, 0.0)
        h = jnp.dot(h, W2_ref[...], preferred_element_type=jnp.float32)
        h = h + b2_ref[0, :]
        o_ref[...] = jnp.maximum(h, 0.0)

    return pl.pallas_call(
        body,
        out_shape=jax.ShapeDtypeStruct((N, Dout), jnp.float32),
    )(x, parts, W1, b1.reshape(1, -1), g.reshape(1, -1), bt.reshape(1, -1),
      W2, b2.reshape(1, -1))


def _head(x1, x2, x3, jk_W, jk_b, batch2d, c1_W, c1_b, cg, cb, c2_W, c2_b):
    def body(x1_ref, x2_ref, x3_ref, jkW_ref, jkb_ref, b_ref, c1W_ref,
             c1b_ref, cg_ref, cb_ref, c2W_ref, c2b_ref, o_ref):
        hcat = jnp.concatenate([x1_ref[...], x2_ref[...], x3_ref[...]], axis=1)
        h = jnp.dot(hcat, jkW_ref[...], preferred_element_type=jnp.float32)
        h = h + jkb_ref[0, :]
        ids = b_ref[0, :]
        onehot = (lax.broadcasted_iota(jnp.int32, (G, N), 0) == ids[None, :])
        pooled = lax.dot(onehot.astype(jnp.float32), h,
                         precision=lax.Precision.HIGHEST,
                         preferred_element_type=jnp.float32)
        z = jnp.dot(pooled, c1W_ref[...], preferred_element_type=jnp.float32)
        z = z + c1b_ref[0, :]
        m = jnp.sum(z, axis=0, keepdims=True) * jnp.float32(1.0 / G)
        d = z - m
        v = jnp.sum(d * d, axis=0, keepdims=True) * jnp.float32(1.0 / G)
        z = d / jnp.sqrt(v + 1e-5) * cg_ref[0, :] + cb_ref[0, :]
        z = jnp.maximum(z, 0.0)
        z = jnp.dot(z, c2W_ref[...], preferred_element_type=jnp.float32)
        o_ref[...] = z + c2b_ref[0, :]

    return pl.pallas_call(
        body,
        out_shape=jax.ShapeDtypeStruct((G, C), jnp.float32),
    )(x1, x2, x3, jk_W, jk_b.reshape(1, -1), batch2d, c1_W,
      c1_b.reshape(1, -1), cg.reshape(1, -1), cb.reshape(1, -1), c2_W,
      c2_b.reshape(1, -1))


def kernel(x, l0_W1, l0_b1, l0_gamma, l0_beta, l0_W2, l0_b2,
           l1_W1, l1_b1, l1_gamma, l1_beta, l1_W2, l1_b2,
           l2_W1, l2_b1, l2_gamma, l2_beta, l2_W2, l2_b2,
           jk_W, jk_b, c1_W, c1_b, c_gamma, c_beta, c2_W, c2_b,
           edge_index, batch):
    src = edge_index[0].astype(jnp.int32)
    dst = edge_index[1].astype(jnp.int32)
    order = jnp.argsort(dst, stable=True).astype(jnp.int32)
    dst_s = dst[order]
    src_s = src[order]
    bnd = jnp.arange(1, NW, dtype=jnp.int32) * EPW
    vals = dst_s[bnd]
    left = jnp.searchsorted(dst_s, vals, side='left').astype(jnp.int32)
    right = jnp.searchsorted(dst_s, vals, side='right').astype(jnp.int32)
    mid = jnp.where(left < bnd, right, bnd)
    starts = jnp.concatenate([jnp.zeros((1,), jnp.int32), mid,
                              jnp.full((1,), E, jnp.int32)])
    lanes = jnp.arange(CAP, dtype=jnp.int32)
    pos = starts[:NW, None] + lanes[None, :]
    valid = pos < starts[1:, None]
    posc = jnp.minimum(pos, E - 1)
    src_f = jnp.where(valid, src_s[posc], lanes[None, :] % N)
    dst_f = jnp.where(valid, dst_s[posc], N + (lanes[None, :] % (N_PAD - N)))
    src_r128 = src_f.reshape(NW, CAP // 64, 64)
    dst_r128 = dst_f.reshape(NW, CAP // 64, 64)
    src_r = src_f.reshape(NW, SPW, LANE)
    dst_r = dst_f.reshape(NW, SPW, LANE)
    zeros128 = jnp.zeros((N_PAD, D), jnp.float32)
    zeros64 = jnp.zeros((N_PAD, H), jnp.float32)
    batch2d = batch.astype(jnp.int32).reshape(1, N)

    parts0 = _seg_sum_128(x, src_r128, dst_r128, zeros128)
    x1 = _gin_mlp(x, parts0, l0_W1, l0_b1, l0_gamma, l0_beta, l0_W2, l0_b2)
    parts1 = _seg_sum_64(x1, src_r, dst_r, zeros64)
    x2 = _gin_mlp(x1, parts1, l1_W1, l1_b1, l1_gamma, l1_beta, l1_W2, l1_b2)
    parts2 = _seg_sum_64(x2, src_r, dst_r, zeros64)
    x3 = _gin_mlp(x2, parts2, l2_W1, l2_b1, l2_gamma, l2_beta, l2_W2, l2_b2)
    return _head(x1, x2, x3, jk_W, jk_b, batch2d, c1_W, c1_b, c_gamma,
                 c_beta, c2_W, c2_b)
